# Initial kernel scaffold; baseline (speedup 1.0000x reference)
import jax
import jax.numpy as jnp
from jax.experimental import pallas as pl


def kernel(x, y, attn):
    raise NotImplementedError("write your pallas kernel here")


# speedup vs baseline: 25.1688x; 25.1688x over previous
import jax
import jax.numpy as jnp
from jax.experimental import pallas as pl

_ROWS = 4096
_COLS = 8192
_XC = 1024
_TOPK = 64
_BLOCK_R = 256
_GRID = _ROWS // _BLOCK_R
_INT_MIN = -(2 ** 31)
_MASK = 0x7FFFFFFF


def _block_kernel(x_ref, y_ref, a_ref, rec_ref, s2_ref, top_ref):
    d = x_ref[...] - y_ref[...]
    rec_ref[...] = jnp.sum(d * d).reshape(1, 1, 1)

    a = a_ref[...]
    sq = a * a
    s2_ref[...] = jnp.sum(sq).reshape(1, 1, 1)

    b = jax.lax.bitcast_convert_type(a, jnp.int32)
    key = jnp.where(b >= 0, b, b ^ _MASK)

    def body(i, u):
        bit = jnp.int32(1) << (jnp.int32(31) - i)
        cand = u | bit
        thresh = cand ^ jnp.int32(_INT_MIN)
        cnt = jnp.sum((key >= thresh).astype(jnp.int32), axis=1,
                      keepdims=True)
        return jnp.where(cnt >= _TOPK, cand, u)

    u0 = jnp.zeros((a.shape[0], 1), jnp.int32)
    u = jax.lax.fori_loop(0, 32, body, u0)
    t_key = u ^ jnp.int32(_INT_MIN)

    gt = key > t_key
    cnt_gt = jnp.sum(gt.astype(jnp.float32), axis=1, keepdims=True)
    ssq_gt = jnp.sum(jnp.where(gt, sq, 0.0), axis=1, keepdims=True)
    t_bits = jnp.where(t_key >= 0, t_key, t_key ^ _MASK)
    t_val = jax.lax.bitcast_convert_type(t_bits, jnp.float32)
    top = ssq_gt + (_TOPK - cnt_gt) * t_val * t_val
    top_ref[...] = jnp.sum(top).reshape(1, 1, 1)


def kernel(x, y, attn):
    rec_p, s2_p, top_p = pl.pallas_call(
        _block_kernel,
        grid=(_GRID,),
        in_specs=[
            pl.BlockSpec((_BLOCK_R, _XC), lambda i: (i, 0)),
            pl.BlockSpec((_BLOCK_R, _XC), lambda i: (i, 0)),
            pl.BlockSpec((_BLOCK_R, _COLS), lambda i: (i, 0)),
        ],
        out_specs=[
            pl.BlockSpec((1, 1, 1), lambda i: (i, 0, 0)),
            pl.BlockSpec((1, 1, 1), lambda i: (i, 0, 0)),
            pl.BlockSpec((1, 1, 1), lambda i: (i, 0, 0)),
        ],
        out_shape=[
            jax.ShapeDtypeStruct((_GRID, 1, 1), jnp.float32),
            jax.ShapeDtypeStruct((_GRID, 1, 1), jnp.float32),
            jax.ShapeDtypeStruct((_GRID, 1, 1), jnp.float32),
        ],
    )(x, y, attn)
    rec = jnp.sum(rec_p) / (_ROWS * _XC)
    attn_loss = (jnp.sum(s2_p) - jnp.sum(top_p)) / (_ROWS * _COLS)
    return rec + 0.5 * attn_loss


# speedup vs baseline: 30.9243x; 1.2287x over previous
import jax
import jax.numpy as jnp
from jax.experimental import pallas as pl

_ROWS = 4096
_COLS = 8192
_XC = 1024
_TOPK = 64
_BLOCK_R = 256
_GRID = _ROWS // _BLOCK_R
_INT_MIN = -(2 ** 31)
_MASK = 0x7FFFFFFF


def _block_kernel(x_ref, y_ref, a_ref, rec_ref, s2_ref, top_ref):
    d = x_ref[...] - y_ref[...]
    rec_ref[...] = jnp.sum(d * d).reshape(1, 1, 1)

    a = a_ref[...]
    sq = a * a
    s2_ref[...] = jnp.sum(sq).reshape(1, 1, 1)

    b = jax.lax.bitcast_convert_type(a, jnp.int32)
    key = jnp.where(b >= 0, b, b ^ _MASK)

    key3 = key.reshape(a.shape[0], _TOPK, _COLS // _TOPK)
    cmax = jnp.max(key3, axis=2)
    lo_key = jnp.min(jnp.min(cmax, axis=1))
    hi_key = jnp.max(jnp.max(cmax, axis=1))
    u_lo = lo_key ^ jnp.int32(_INT_MIN)
    u_hi = hi_key ^ jnp.int32(_INT_MIN)
    diff = u_lo ^ u_hi
    s = diff
    s = s | jax.lax.shift_right_logical(s, 1)
    s = s | jax.lax.shift_right_logical(s, 2)
    s = s | jax.lax.shift_right_logical(s, 4)
    s = s | jax.lax.shift_right_logical(s, 8)
    s = s | jax.lax.shift_right_logical(s, 16)
    sp1 = s + 1
    exp = jax.lax.shift_right_logical(
        jax.lax.bitcast_convert_type(sp1.astype(jnp.float32), jnp.int32),
        23) - 127
    nbits = jnp.where(s < 0, jnp.int32(32),
                      jnp.where(s == 0, jnp.int32(0), exp))

    u_start = u_hi & ~s

    def body(i, u):
        bit = jnp.int32(1) << (nbits - 1 - i)
        cand = u | bit
        thresh = cand ^ jnp.int32(_INT_MIN)
        cnt = jnp.sum((key >= thresh).astype(jnp.int32), axis=1,
                      keepdims=True)
        return jnp.where(cnt >= _TOPK, cand, u)

    u0 = jnp.full((a.shape[0], 1), u_start, jnp.int32)
    u = jax.lax.fori_loop(0, nbits, body, u0)
    t_key = u ^ jnp.int32(_INT_MIN)

    gt = key > t_key
    cnt_gt = jnp.sum(gt.astype(jnp.float32), axis=1, keepdims=True)
    ssq_gt = jnp.sum(jnp.where(gt, sq, 0.0), axis=1, keepdims=True)
    t_bits = jnp.where(t_key >= 0, t_key, t_key ^ _MASK)
    t_val = jax.lax.bitcast_convert_type(t_bits, jnp.float32)
    top = ssq_gt + (_TOPK - cnt_gt) * t_val * t_val
    top_ref[...] = jnp.sum(top).reshape(1, 1, 1)


def kernel(x, y, attn):
    rec_p, s2_p, top_p = pl.pallas_call(
        _block_kernel,
        grid=(_GRID,),
        in_specs=[
            pl.BlockSpec((_BLOCK_R, _XC), lambda i: (i, 0)),
            pl.BlockSpec((_BLOCK_R, _XC), lambda i: (i, 0)),
            pl.BlockSpec((_BLOCK_R, _COLS), lambda i: (i, 0)),
        ],
        out_specs=[
            pl.BlockSpec((1, 1, 1), lambda i: (i, 0, 0)),
            pl.BlockSpec((1, 1, 1), lambda i: (i, 0, 0)),
            pl.BlockSpec((1, 1, 1), lambda i: (i, 0, 0)),
        ],
        out_shape=[
            jax.ShapeDtypeStruct((_GRID, 1, 1), jnp.float32),
            jax.ShapeDtypeStruct((_GRID, 1, 1), jnp.float32),
            jax.ShapeDtypeStruct((_GRID, 1, 1), jnp.float32),
        ],
    )(x, y, attn)
    rec = jnp.sum(rec_p) / (_ROWS * _XC)
    attn_loss = (jnp.sum(s2_p) - jnp.sum(top_p)) / (_ROWS * _COLS)
    return rec + 0.5 * attn_loss


# speedup vs baseline: 38.8728x; 1.2570x over previous
import jax
import jax.numpy as jnp
from jax.experimental import pallas as pl

_ROWS = 4096
_COLS = 8192
_XC = 1024
_TOPK = 64
_BLOCK_R = 256
_GRID = _ROWS // _BLOCK_R
_INT_MIN = -(2 ** 31)
_MASK = 0x7FFFFFFF


def _count16(mask16):
    m = mask16
    while m.shape[1] > 128:
        h = m.shape[1] // 2
        m = m[:, :h] + m[:, h:]
    return jnp.sum(m.astype(jnp.int32), axis=1, keepdims=True)


def _block_kernel(x_ref, y_ref, a_ref, rec_ref, s2_ref, top_ref):
    d = x_ref[...] - y_ref[...]
    rec_ref[...] = jnp.sum(d * d).reshape(1, 1, 1)

    a = a_ref[...]
    sq = a * a
    s2_ref[...] = jnp.sum(sq).reshape(1, 1, 1)

    b = jax.lax.bitcast_convert_type(a, jnp.int32)
    key = jnp.where(b >= 0, b, b ^ _MASK)

    key3 = key.reshape(a.shape[0], _TOPK, _COLS // _TOPK)
    cmax = jnp.max(key3, axis=2)
    lo_key = jnp.min(jnp.min(cmax, axis=1))
    hi_key = jnp.max(jnp.max(cmax, axis=1))
    u_lo = lo_key ^ jnp.int32(_INT_MIN)
    u_hi = hi_key ^ jnp.int32(_INT_MIN)
    diff = u_lo ^ u_hi
    s = diff
    s = s | jax.lax.shift_right_logical(s, 1)
    s = s | jax.lax.shift_right_logical(s, 2)
    s = s | jax.lax.shift_right_logical(s, 4)
    s = s | jax.lax.shift_right_logical(s, 8)
    s = s | jax.lax.shift_right_logical(s, 16)
    sp1 = s + 1
    exp = jax.lax.shift_right_logical(
        jax.lax.bitcast_convert_type(sp1.astype(jnp.float32), jnp.int32),
        23) - 127
    nbits = jnp.where(s < 0, jnp.int32(32),
                      jnp.where(s == 0, jnp.int32(0), exp))

    u_start = u_hi & ~s

    k16 = (key >> 16).astype(jnp.int16)
    nh = jnp.maximum(nbits - 16, 0)
    uh_start = jax.lax.shift_right_logical(u_start, 16)

    def body_hi(i, uh):
        bit = jnp.int32(1) << (nh - 1 - i)
        cand = uh | bit
        t16c = (cand ^ 0x8000).astype(jnp.int16)
        cnt = _count16((k16 >= t16c).astype(jnp.int16))
        return jnp.where(cnt >= _TOPK, cand, uh)

    uh0 = jnp.full((a.shape[0], 1), uh_start, jnp.int32)
    uh = jax.lax.fori_loop(0, nh, body_hi, uh0)
    t16 = (uh ^ 0x8000).astype(jnp.int16)

    c_gt_hi = _count16((k16 > t16).astype(jnp.int16))
    r = _TOPK - c_gt_hi

    l16s = key.astype(jnp.int16) ^ jnp.int16(-(2 ** 15))
    m16 = jnp.where(k16 == t16, l16s, jnp.int16(-(2 ** 15)))
    nl = jnp.minimum(nbits, 16)
    ul_start = u_start & 0xFFFF & ~((jnp.int32(1) << nl) - 1)

    def body_lo(i, ul):
        bit = jnp.int32(1) << (nl - 1 - i)
        cand = ul | bit
        ts = (cand ^ 0x8000).astype(jnp.int16)
        cnt = _count16((m16 >= ts).astype(jnp.int16))
        return jnp.where(cnt >= r, cand, ul)

    ul0 = jnp.full((a.shape[0], 1), ul_start, jnp.int32)
    ul = jax.lax.fori_loop(0, nl, body_lo, ul0)

    t_key = (jax.lax.convert_element_type(t16, jnp.int32) << 16) | ul

    gt = key > t_key
    cnt_gt = jnp.sum(gt.astype(jnp.float32), axis=1, keepdims=True)
    ssq_gt = jnp.sum(jnp.where(gt, sq, 0.0), axis=1, keepdims=True)
    t_bits = jnp.where(t_key >= 0, t_key, t_key ^ _MASK)
    t_val = jax.lax.bitcast_convert_type(t_bits, jnp.float32)
    top = ssq_gt + (_TOPK - cnt_gt) * t_val * t_val
    top_ref[...] = jnp.sum(top).reshape(1, 1, 1)


def kernel(x, y, attn):
    rec_p, s2_p, top_p = pl.pallas_call(
        _block_kernel,
        grid=(_GRID,),
        in_specs=[
            pl.BlockSpec((_BLOCK_R, _XC), lambda i: (i, 0)),
            pl.BlockSpec((_BLOCK_R, _XC), lambda i: (i, 0)),
            pl.BlockSpec((_BLOCK_R, _COLS), lambda i: (i, 0)),
        ],
        out_specs=[
            pl.BlockSpec((1, 1, 1), lambda i: (i, 0, 0)),
            pl.BlockSpec((1, 1, 1), lambda i: (i, 0, 0)),
            pl.BlockSpec((1, 1, 1), lambda i: (i, 0, 0)),
        ],
        out_shape=[
            jax.ShapeDtypeStruct((_GRID, 1, 1), jnp.float32),
            jax.ShapeDtypeStruct((_GRID, 1, 1), jnp.float32),
            jax.ShapeDtypeStruct((_GRID, 1, 1), jnp.float32),
        ],
    )(x, y, attn)
    rec = jnp.sum(rec_p) / (_ROWS * _XC)
    attn_loss = (jnp.sum(s2_p) - jnp.sum(top_p)) / (_ROWS * _COLS)
    return rec + 0.5 * attn_loss


# speedup vs baseline: 39.4436x; 1.0147x over previous
import jax
import jax.numpy as jnp
from jax.experimental import pallas as pl

_ROWS = 4096
_COLS = 8192
_XC = 1024
_TOPK = 64
_BLOCK_R = 256
_GRID = _ROWS // _BLOCK_R
_INT_MIN = -(2 ** 31)
_MASK = 0x7FFFFFFF


def _count16(mask16):
    m = mask16
    while m.shape[1] > 128:
        h = m.shape[1] // 2
        m = m[:, :h] + m[:, h:]
    return jnp.sum(m.astype(jnp.int32), axis=1, keepdims=True)


def _block_kernel(x_ref, y_ref, a_ref, rec_ref, s2_ref, top_ref):
    d = x_ref[...] - y_ref[...]
    rec_ref[...] = jnp.sum(d * d).reshape(1, 1, 1)

    a = a_ref[...]
    sq = a * a
    s2_ref[...] = jnp.sum(sq).reshape(1, 1, 1)

    b = jax.lax.bitcast_convert_type(a, jnp.int32)
    key = jnp.where(b >= 0, b, b ^ _MASK)

    key3 = key.reshape(a.shape[0], _COLS // 128, 128)
    cmax = jnp.max(key3, axis=1)
    lo_key = jnp.min(cmax)
    hi_key = jnp.max(cmax)
    u_lo = lo_key ^ jnp.int32(_INT_MIN)
    u_hi = hi_key ^ jnp.int32(_INT_MIN)
    diff = u_lo ^ u_hi
    s = diff
    s = s | jax.lax.shift_right_logical(s, 1)
    s = s | jax.lax.shift_right_logical(s, 2)
    s = s | jax.lax.shift_right_logical(s, 4)
    s = s | jax.lax.shift_right_logical(s, 8)
    s = s | jax.lax.shift_right_logical(s, 16)
    sp1 = s + 1
    exp = jax.lax.shift_right_logical(
        jax.lax.bitcast_convert_type(sp1.astype(jnp.float32), jnp.int32),
        23) - 127
    nbits = jnp.where(s < 0, jnp.int32(32),
                      jnp.where(s == 0, jnp.int32(0), exp))

    u_start = u_hi & ~s

    k16 = (key >> 16).astype(jnp.int16)
    nh = jnp.maximum(nbits - 16, 0)
    uh_start = jax.lax.shift_right_logical(u_start, 16)

    def body_hi(i, uh):
        bit = jnp.int32(1) << (nh - 1 - i)
        cand = uh | bit
        t16c = (cand ^ 0x8000).astype(jnp.int16)
        cnt = _count16((k16 >= t16c).astype(jnp.int16))
        return jnp.where(cnt >= _TOPK, cand, uh)

    uh0 = jnp.full((a.shape[0], 1), uh_start, jnp.int32)
    uh = jax.lax.fori_loop(0, nh, body_hi, uh0)
    t16 = (uh ^ 0x8000).astype(jnp.int16)

    c_gt_hi = _count16((k16 > t16).astype(jnp.int16))
    r = _TOPK - c_gt_hi

    l16s = key.astype(jnp.int16) ^ jnp.int16(-(2 ** 15))
    m16 = jnp.where(k16 == t16, l16s, jnp.int16(-(2 ** 15)))
    nl = jnp.minimum(nbits, 16)
    ul_start = u_start & 0xFFFF & ~((jnp.int32(1) << nl) - 1)

    def body_lo(i, ul):
        bit = jnp.int32(1) << (nl - 1 - i)
        cand = ul | bit
        ts = (cand ^ 0x8000).astype(jnp.int16)
        cnt = _count16((m16 >= ts).astype(jnp.int16))
        return jnp.where(cnt >= r, cand, ul)

    ul0 = jnp.full((a.shape[0], 1), ul_start, jnp.int32)
    ul = jax.lax.fori_loop(0, nl, body_lo, ul0)

    t_key = (jax.lax.convert_element_type(t16, jnp.int32) << 16) | ul

    gt = key > t_key
    cnt_gt = jnp.sum(gt.astype(jnp.float32), axis=1, keepdims=True)
    ssq_gt = jnp.sum(jnp.where(gt, sq, 0.0), axis=1, keepdims=True)
    t_bits = jnp.where(t_key >= 0, t_key, t_key ^ _MASK)
    t_val = jax.lax.bitcast_convert_type(t_bits, jnp.float32)
    top = ssq_gt + (_TOPK - cnt_gt) * t_val * t_val
    top_ref[...] = jnp.sum(top).reshape(1, 1, 1)


def kernel(x, y, attn):
    rec_p, s2_p, top_p = pl.pallas_call(
        _block_kernel,
        grid=(_GRID,),
        in_specs=[
            pl.BlockSpec((_BLOCK_R, _XC), lambda i: (i, 0)),
            pl.BlockSpec((_BLOCK_R, _XC), lambda i: (i, 0)),
            pl.BlockSpec((_BLOCK_R, _COLS), lambda i: (i, 0)),
        ],
        out_specs=[
            pl.BlockSpec((1, 1, 1), lambda i: (i, 0, 0)),
            pl.BlockSpec((1, 1, 1), lambda i: (i, 0, 0)),
            pl.BlockSpec((1, 1, 1), lambda i: (i, 0, 0)),
        ],
        out_shape=[
            jax.ShapeDtypeStruct((_GRID, 1, 1), jnp.float32),
            jax.ShapeDtypeStruct((_GRID, 1, 1), jnp.float32),
            jax.ShapeDtypeStruct((_GRID, 1, 1), jnp.float32),
        ],
    )(x, y, attn)
    rec = jnp.sum(rec_p) / (_ROWS * _XC)
    attn_loss = (jnp.sum(s2_p) - jnp.sum(top_p)) / (_ROWS * _COLS)
    return rec + 0.5 * attn_loss
